# Initial kernel scaffold; baseline (speedup 1.0000x reference)
import functools

import jax
import jax.numpy as jnp
from jax import lax
from jax.experimental import pallas as pl
from jax.experimental.pallas import tpu as pltpu
from jax.experimental.pallas import tpu_sc as plsc

_NC = 2
_NS = 16
_L = 16
_NW = _NC * _NS


def _build_table(graph_embedding, semantic_weight, N, D):
    BN = 500 if N % 500 == 0 else 8
    assert N % BN == 0
    nb = N // BN

    def body(sw_ref, x_ref, o_ref):
        g = pl.program_id(0)
        s0 = sw_ref[0, 0]
        extra = (
            jnp.where(g == 1, sw_ref[0, 1], 0.0)
            + jnp.where(g == 2, sw_ref[0, 2], 0.0)
            + jnp.where(g == 3, sw_ref[0, 3], 0.0)
            + jnp.where(g == 4, sw_ref[0, 4], 0.0)
        )
        o_ref[...] = x_ref[...] * (s0 + extra)

    return pl.pallas_call(
        body,
        grid=(5, nb),
        in_specs=[
            pl.BlockSpec(memory_space=pltpu.SMEM),
            pl.BlockSpec((BN, D), lambda g, rb: (rb, 0)),
        ],
        out_specs=pl.BlockSpec((BN, D), lambda g, rb: (g * nb + rb, 0)),
        out_shape=jax.ShapeDtypeStruct((5 * N, D), jnp.float32),
    )(semantic_weight, graph_embedding)


def _combine(parts, weight, N, D):
    BN = 500 if N % 500 == 0 else 8
    nb = N // BN

    def body(a_ref, b_ref, w_ref, o_ref):
        o_ref[...] = (a_ref[...] + b_ref[...]) * w_ref[...]

    return pl.pallas_call(
        body,
        grid=(nb,),
        in_specs=[
            pl.BlockSpec((BN, D), lambda i: (i, 0)),
            pl.BlockSpec((BN, D), lambda i, _nb=nb: (i + _nb, 0)),
            pl.BlockSpec((1, D), lambda i: (0, 0)),
        ],
        out_specs=pl.BlockSpec((BN, D), lambda i: (i, 0)),
        out_shape=jax.ShapeDtypeStruct((N, D), jnp.float32),
    )(parts, parts, weight)


def _sc_gather_scatter(table, src, dst, ef, zblock, N, D, E):
    assert E % _NW == 0
    EPW = E // _NW
    CH = 128
    NCH = -(-EPW // CH)
    if NCH % 2:
        NCH += 1
    PADE = NCH * CH
    NDUM = N
    NPAD = N + _L
    RPS = NPAD // _NS
    OPS = N // _NS
    assert N % _NS == 0 and D % _L == 0

    mesh = plsc.VectorSubcoreMesh(
        core_axis_name="c", subcore_axis_name="s",
        num_cores=_NC, num_subcores=_NS,
    )

    @functools.partial(
        pl.kernel,
        out_type=jax.ShapeDtypeStruct((_NC * N, D), jnp.float32),
        mesh=mesh,
        scratch_types=[
            pltpu.VMEM((PADE,), jnp.int32),
            pltpu.VMEM((PADE,), jnp.int32),
            pltpu.VMEM((PADE,), jnp.int32),
            pltpu.VMEM((NCH, CH), jnp.int32),
            pltpu.VMEM((NCH, CH), jnp.int32),
            pltpu.VMEM((CH, D), jnp.float32),
            pltpu.VMEM_SHARED((NPAD, D), jnp.float32),
            pltpu.SemaphoreType.DMA,
        ],
    )
    def sc_k(table_ref, src_ref, dst_ref, ef_ref, zb_ref, parts_ref,
             srcv, dstv, efv, gidx, didx, rows0, acc, sem):
        c = lax.axis_index("c")
        s = lax.axis_index("s")
        wid = s * _NC + c
        base = wid * EPW

        pltpu.sync_copy(src_ref.at[pl.ds(base, EPW)], srcv.at[pl.ds(0, EPW)])
        pltpu.sync_copy(dst_ref.at[pl.ds(base, EPW)], dstv.at[pl.ds(0, EPW)])
        pltpu.sync_copy(ef_ref.at[pl.ds(base, EPW)], efv.at[pl.ds(0, EPW)])

        pltpu.sync_copy(zb_ref, rows0)
        for k in range(-(-RPS // CH)):
            start = jnp.minimum(s * RPS + k * CH, NPAD - CH)
            pltpu.sync_copy(rows0, acc.at[pl.ds(start, CH)])

        iot = lax.iota(jnp.int32, _L)

        def comp(j, carry):
            for q in range(CH // _L):
                off = j * CH + q * _L
                sv = srcv[pl.ds(off, _L)]
                ev = efv[pl.ds(off, _L)]
                dv = dstv[pl.ds(off, _L)]
                real = (off + iot) < EPW
                g = jnp.where((ev & 1) == 1, 0, (ev >> 1) + 1)
                gidx[j, pl.ds(q * _L, _L)] = jnp.where(real, sv + N * g, 0)
                didx[j, pl.ds(q * _L, _L)] = jnp.where(real, dv, NDUM)
            return carry

        lax.fori_loop(0, NCH, comp, 0)

        plsc.subcore_barrier()

        def mloop(j, carry):
            pltpu.async_copy(table_ref.at[gidx.at[j]], rows0, sem).wait()
            pltpu.sync_copy(rows0, acc.at[didx.at[j]], add=True)
            return carry

        lax.fori_loop(0, NCH, mloop, 0)

        plsc.subcore_barrier()

        pltpu.sync_copy(
            acc.at[pl.ds(s * OPS, OPS)],
            parts_ref.at[pl.ds(c * N + s * OPS, OPS)],
        )

    return sc_k(table, src, dst, ef, zblock)


@jax.jit
def kernel(graph_embedding, edge_index, e_feat, weight, semantic_weight):
    N, D = graph_embedding.shape
    E = edge_index.shape[1]

    src = edge_index[0]
    dst = edge_index[1]
    zblock = jnp.zeros((128, D), dtype=jnp.float32)

    table = _build_table(graph_embedding, semantic_weight, N, D)
    parts = _sc_gather_scatter(table, src, dst, e_feat, zblock, N, D, E)
    return _combine(parts, weight, N, D)


# speedup vs baseline: 3.5305x; 3.5305x over previous
import functools

import jax
import jax.numpy as jnp
from jax import lax
from jax.experimental import pallas as pl
from jax.experimental.pallas import tpu as pltpu
from jax.experimental.pallas import tpu_sc as plsc

_NC = 2
_NS = 16
_L = 16
_NW = _NC * _NS


def _build_table(graph_embedding, semantic_weight, N, D):
    BN = 1000 if N % 1000 == 0 else 8
    assert N % BN == 0
    nb = N // BN

    def body(sw_ref, x_ref, o_ref):
        g = pl.program_id(0)
        s0 = sw_ref[0, 0]
        extra = (
            jnp.where(g == 1, sw_ref[0, 1], 0.0)
            + jnp.where(g == 2, sw_ref[0, 2], 0.0)
            + jnp.where(g == 3, sw_ref[0, 3], 0.0)
            + jnp.where(g == 4, sw_ref[0, 4], 0.0)
        )
        o_ref[...] = x_ref[...] * (s0 + extra)

    return pl.pallas_call(
        body,
        grid=(5, nb),
        in_specs=[
            pl.BlockSpec(memory_space=pltpu.SMEM),
            pl.BlockSpec((BN, D), lambda g, rb: (rb, 0)),
        ],
        out_specs=pl.BlockSpec((BN, D), lambda g, rb: (g * nb + rb, 0)),
        out_shape=jax.ShapeDtypeStruct((5 * N, D), jnp.float32),
    )(semantic_weight, graph_embedding)


def _combine(parts, weight, N, D):
    BN = 1000 if N % 1000 == 0 else 8
    nb = N // BN

    def body(a_ref, b_ref, w_ref, o_ref):
        o_ref[...] = (a_ref[...] + b_ref[...]) * w_ref[...]

    return pl.pallas_call(
        body,
        grid=(nb,),
        in_specs=[
            pl.BlockSpec((BN, D), lambda i: (i, 0)),
            pl.BlockSpec((BN, D), lambda i, _nb=nb: (i + _nb, 0)),
            pl.BlockSpec((1, D), lambda i: (0, 0)),
        ],
        out_specs=pl.BlockSpec((BN, D), lambda i: (i, 0)),
        out_shape=jax.ShapeDtypeStruct((N, D), jnp.float32),
    )(parts, parts, weight)


def _sc_gather_scatter(table, src_p, dst_p, ef_p, zblock, N, D, EP):
    CH = 128
    GC = 8
    PADE = EP // _NW
    assert PADE % (GC * CH) == 0
    NG = PADE // (GC * CH)
    NPAD = N + _L
    RPS = (-(-NPAD // _NS) + 7) // 8 * 8
    OPS = (-(-N // _NS) + 7) // 8 * 8
    assert N % 8 == 0 and NPAD % 8 == 0 and D % _L == 0

    mesh = plsc.VectorSubcoreMesh(
        core_axis_name="c", subcore_axis_name="s",
        num_cores=_NC, num_subcores=_NS,
    )

    @functools.partial(
        pl.kernel,
        out_type=jax.ShapeDtypeStruct((_NC * N, D), jnp.float32),
        mesh=mesh,
        scratch_types=[
            pltpu.VMEM((GC * CH,), jnp.int32),
            pltpu.VMEM((GC * CH,), jnp.int32),
            pltpu.VMEM((GC * CH,), jnp.int32),
            pltpu.VMEM((GC, CH), jnp.int32),
            pltpu.VMEM((GC, CH), jnp.int32),
            pltpu.VMEM((CH, D), jnp.float32),
            pltpu.VMEM_SHARED((NPAD, D), jnp.float32),
            pltpu.SemaphoreType.DMA,
        ],
    )
    def sc_k(table_ref, src_ref, dst_ref, ef_ref, zb_ref, parts_ref,
             srcg, dstg, efg, gidx, didx, rows0, acc, sem):
        c = lax.axis_index("c")
        s = lax.axis_index("s")
        wid = s * _NC + c
        base = wid * PADE

        pltpu.sync_copy(zb_ref, rows0)
        for k in range(-(-RPS // CH)):
            start = jnp.minimum(s * RPS + k * CH, NPAD - CH)
            pltpu.sync_copy(rows0, acc.at[pl.ds(start, CH)])

        plsc.subcore_barrier()

        def group(t, carry):
            gbase = base + t * (GC * CH)
            pltpu.sync_copy(src_ref.at[pl.ds(gbase, GC * CH)], srcg)
            pltpu.sync_copy(dst_ref.at[pl.ds(gbase, GC * CH)], dstg)
            pltpu.sync_copy(ef_ref.at[pl.ds(gbase, GC * CH)], efg)
            for q in range(GC):
                for k in range(CH // _L):
                    off = q * CH + k * _L
                    sv = srcg[pl.ds(off, _L)]
                    ev = efg[pl.ds(off, _L)]
                    dv = dstg[pl.ds(off, _L)]
                    g = jnp.where((ev & 1) == 1, 0, (ev >> 1) + 1)
                    gidx[q, pl.ds(k * _L, _L)] = sv + N * g
                    didx[q, pl.ds(k * _L, _L)] = dv
            for q in range(GC):
                pltpu.async_copy(table_ref.at[gidx.at[q]], rows0, sem).wait()
                pltpu.sync_copy(rows0, acc.at[didx.at[q]], add=True)
            return carry

        lax.fori_loop(0, NG, group, 0)

        plsc.subcore_barrier()

        ostart = jnp.minimum(s * OPS, N - OPS)
        pltpu.sync_copy(
            acc.at[pl.ds(ostart, OPS)],
            parts_ref.at[pl.ds(c * N + ostart, OPS)],
        )

    return sc_k(table, src_p, dst_p, ef_p, zblock)


@jax.jit
def kernel(graph_embedding, edge_index, e_feat, weight, semantic_weight):
    N, D = graph_embedding.shape
    E = edge_index.shape[1]

    CHUNK = _NW * 8 * 128
    EP = -(-E // CHUNK) * CHUNK
    pad = EP - E
    src_p = jnp.concatenate([edge_index[0], jnp.zeros((pad,), jnp.int32)])
    dst_p = jnp.concatenate([edge_index[1], jnp.full((pad,), N, jnp.int32)])
    ef_p = jnp.concatenate([e_feat, jnp.ones((pad,), jnp.int32)])
    zblock = jnp.zeros((128, D), dtype=jnp.float32)

    table = _build_table(graph_embedding, semantic_weight, N, D)
    parts = _sc_gather_scatter(table, src_p, dst_p, ef_p, zblock, N, D, EP)
    return _combine(parts, weight, N, D)


# speedup vs baseline: 3.9122x; 1.1081x over previous
import functools

import jax
import jax.numpy as jnp
from jax import lax
from jax.experimental import pallas as pl
from jax.experimental.pallas import tpu as pltpu
from jax.experimental.pallas import tpu_sc as plsc

_NC = 2
_NS = 16
_L = 16
_NW = _NC * _NS


def _build_table(graph_embedding, semantic_weight, N, D):
    BN = 1000 if N % 1000 == 0 else 8
    assert N % BN == 0
    nb = N // BN

    def body(sw_ref, x_ref, o_ref):
        g = pl.program_id(0)
        s0 = sw_ref[0, 0]
        extra = (
            jnp.where(g == 1, sw_ref[0, 1], 0.0)
            + jnp.where(g == 2, sw_ref[0, 2], 0.0)
            + jnp.where(g == 3, sw_ref[0, 3], 0.0)
            + jnp.where(g == 4, sw_ref[0, 4], 0.0)
        )
        o_ref[...] = x_ref[...] * (s0 + extra)

    return pl.pallas_call(
        body,
        grid=(5, nb),
        in_specs=[
            pl.BlockSpec(memory_space=pltpu.SMEM),
            pl.BlockSpec((BN, D), lambda g, rb: (rb, 0)),
        ],
        out_specs=pl.BlockSpec((BN, D), lambda g, rb: (g * nb + rb, 0)),
        out_shape=jax.ShapeDtypeStruct((5 * N, D), jnp.float32),
    )(semantic_weight, graph_embedding)


def _combine(parts, weight, N, D):
    BN = 1000 if N % 1000 == 0 else 8
    nb = N // BN

    def body(a_ref, b_ref, w_ref, o_ref):
        o_ref[...] = (a_ref[...] + b_ref[...]) * w_ref[...]

    return pl.pallas_call(
        body,
        grid=(nb,),
        in_specs=[
            pl.BlockSpec((BN, D), lambda i: (i, 0)),
            pl.BlockSpec((BN, D), lambda i, _nb=nb: (i + _nb, 0)),
            pl.BlockSpec((1, D), lambda i: (0, 0)),
        ],
        out_specs=pl.BlockSpec((BN, D), lambda i: (i, 0)),
        out_shape=jax.ShapeDtypeStruct((N, D), jnp.float32),
    )(parts, parts, weight)


def _sc_gather_scatter(table, src_p, dst_p, ef_p, zblock, N, D, EP):
    CH = 128
    GC = 8
    PADE = EP // _NW
    assert PADE % (GC * CH) == 0
    NG = PADE // (GC * CH)
    NPAD = N + _L
    RPS = (-(-NPAD // _NS) + 7) // 8 * 8
    OPS = (-(-N // _NS) + 7) // 8 * 8
    assert N % 8 == 0 and NPAD % 8 == 0 and D % _L == 0

    mesh = plsc.VectorSubcoreMesh(
        core_axis_name="c", subcore_axis_name="s",
        num_cores=_NC, num_subcores=_NS,
    )

    @functools.partial(
        pl.kernel,
        out_type=jax.ShapeDtypeStruct((_NC * N, D), jnp.float32),
        mesh=mesh,
        scratch_types=[
            pltpu.VMEM((GC * CH,), jnp.int32),
            pltpu.VMEM((GC * CH,), jnp.int32),
            pltpu.VMEM((GC * CH,), jnp.int32),
            pltpu.VMEM((GC, CH), jnp.int32),
            pltpu.VMEM((GC, CH), jnp.int32),
            pltpu.VMEM((CH, D), jnp.float32),
            pltpu.VMEM((CH, D), jnp.float32),
            pltpu.VMEM_SHARED((NPAD, D), jnp.float32),
            pltpu.SemaphoreType.DMA,
            pltpu.SemaphoreType.DMA,
            pltpu.SemaphoreType.DMA,
            pltpu.SemaphoreType.DMA,
        ],
    )
    def sc_k(table_ref, src_ref, dst_ref, ef_ref, zb_ref, parts_ref,
             srcg, dstg, efg, gidx, didx, rows0, rows1, acc,
             gsem0, gsem1, ssem0, ssem1):
        rows = (rows0, rows1)
        gsem = (gsem0, gsem1)
        ssem = (ssem0, ssem1)
        c = lax.axis_index("c")
        s = lax.axis_index("s")
        wid = s * _NC + c
        base = wid * PADE

        pltpu.sync_copy(zb_ref, rows0)
        for k in range(-(-RPS // CH)):
            start = jnp.minimum(s * RPS + k * CH, NPAD - CH)
            pltpu.sync_copy(rows0, acc.at[pl.ds(start, CH)])

        plsc.subcore_barrier()

        def group(t, carry):
            gbase = base + t * (GC * CH)
            pltpu.sync_copy(src_ref.at[pl.ds(gbase, GC * CH)], srcg)
            pltpu.sync_copy(dst_ref.at[pl.ds(gbase, GC * CH)], dstg)
            pltpu.sync_copy(ef_ref.at[pl.ds(gbase, GC * CH)], efg)
            @pl.when(t > 0)
            def _():
                for b in range(2):
                    pltpu.make_async_copy(
                        rows[b], acc.at[didx.at[GC - 2 + b]], ssem[b]
                    ).wait()

            for q in range(GC):
                for k in range(CH // _L):
                    off = q * CH + k * _L
                    sv = srcg[pl.ds(off, _L)]
                    ev = efg[pl.ds(off, _L)]
                    dv = dstg[pl.ds(off, _L)]
                    g = jnp.where((ev & 1) == 1, 0, (ev >> 1) + 1)
                    gidx[q, pl.ds(k * _L, _L)] = sv + N * g
                    didx[q, pl.ds(k * _L, _L)] = dv

            gd = [None, None]
            sd = [None, None]
            for q in range(GC):
                b = q % 2
                if q >= 2:
                    sd[b].wait()
                gd[b] = pltpu.async_copy(
                    table_ref.at[gidx.at[q]], rows[b], gsem[b])
                if q >= 1:
                    gd[1 - b].wait()
                    sd[1 - b] = pltpu.async_copy(
                        rows[1 - b], acc.at[didx.at[q - 1]], ssem[1 - b],
                        add=True)
            gd[(GC - 1) % 2].wait()
            sd[(GC - 1) % 2] = pltpu.async_copy(
                rows[(GC - 1) % 2], acc.at[didx.at[GC - 1]],
                ssem[(GC - 1) % 2], add=True)
            return carry

        lax.fori_loop(0, NG, group, 0)

        for b in range(2):
            pltpu.make_async_copy(
                rows[b], acc.at[didx.at[GC - 2 + b]], ssem[b]
            ).wait()

        plsc.subcore_barrier()

        ostart = jnp.minimum(s * OPS, N - OPS)
        pltpu.sync_copy(
            acc.at[pl.ds(ostart, OPS)],
            parts_ref.at[pl.ds(c * N + ostart, OPS)],
        )

    return sc_k(table, src_p, dst_p, ef_p, zblock)


@jax.jit
def kernel(graph_embedding, edge_index, e_feat, weight, semantic_weight):
    N, D = graph_embedding.shape
    E = edge_index.shape[1]

    CHUNK = _NW * 8 * 128
    EP = -(-E // CHUNK) * CHUNK
    pad = EP - E
    src_p = jnp.concatenate([edge_index[0], jnp.zeros((pad,), jnp.int32)])
    dst_p = jnp.concatenate([edge_index[1], jnp.full((pad,), N, jnp.int32)])
    ef_p = jnp.concatenate([e_feat, jnp.ones((pad,), jnp.int32)])
    zblock = jnp.zeros((128, D), dtype=jnp.float32)

    table = _build_table(graph_embedding, semantic_weight, N, D)
    parts = _sc_gather_scatter(table, src_p, dst_p, ef_p, zblock, N, D, EP)
    return _combine(parts, weight, N, D)
